# baseline (speedup 1.0000x reference)
import functools
import math

import jax
import jax.numpy as jnp
import numpy as np
from jax import lax
from jax.experimental import pallas as pl
from jax.experimental.pallas import tpu as pltpu
from jax.experimental.pallas import tpu_sc as plsc

D_MODEL = 768
POS_LEN = 2048
_SCALE = math.sqrt(float(D_MODEL))

NC, NS = 2, 16
NW = NC * NS
LANES = 16
SUB = 32
NBUF = 3


def _positional_encoding_np(length, depth):
    half = depth / 2
    positions = np.arange(length)[:, np.newaxis]
    depths = np.arange(half)[np.newaxis, :] / half
    angle_rates = 1 / 10000 ** depths
    angle_rads = positions * angle_rates
    return np.concatenate(
        [np.sin(angle_rads), np.cos(angle_rads)], axis=-1
    ).astype(np.float32)


_POS_NP = _positional_encoding_np(POS_LEN, D_MODEL)


@functools.partial(jax.jit, static_argnames=())
def _sc_embed(x, pos, table):
    b_seq, t_seq = x.shape
    d = table.shape[1]
    t_per_w = t_seq // NW
    spw = t_per_w // SUB
    nsub = b_seq * spw
    j_iters = d // LANES
    mesh = plsc.VectorSubcoreMesh(core_axis_name="c", subcore_axis_name="s")

    @functools.partial(
        pl.kernel,
        out_type=jax.ShapeDtypeStruct((b_seq, t_seq, d), jnp.float32),
        mesh=mesh,
        scratch_types=[
            pltpu.VMEM((b_seq, t_per_w), jnp.int32),
            pltpu.VMEM((t_per_w, d), jnp.float32),
        ]
        + [pltpu.VMEM((SUB, d), jnp.float32) for _ in range(NBUF)]
        + [
            pltpu.SemaphoreType.DMA,
            pltpu.SemaphoreType.DMA,
            pltpu.SemaphoreType.DMA,
        ],
    )
    def k(x_hbm, pos_hbm, table_hbm, out_hbm, idx_v, pos_v, *rest):
        bufs, (gsem, ssem, psem) = rest[:NBUF], rest[NBUF:]
        w = lax.axis_index("s") * NC + lax.axis_index("c")
        t0 = w * t_per_w
        for b in range(b_seq):
            pltpu.sync_copy(x_hbm.at[b, pl.ds(t0, t_per_w)], idx_v.at[b])
        pcopy = pltpu.async_copy(pos_hbm.at[pl.ds(t0, t_per_w)], pos_v, psem)

        def idx_ref(i):
            return idx_v.at[i // spw, pl.ds((i % spw) * SUB, SUB)]

        gathers = [None] * nsub
        stores = [None] * nsub
        gathers[0] = pltpu.async_copy(table_hbm.at[idx_ref(0)], bufs[0], gsem)
        for i in range(nsub):
            if i + 1 < nsub:
                if i + 1 >= NBUF:
                    stores[i + 1 - NBUF].wait()
                gathers[i + 1] = pltpu.async_copy(
                    table_hbm.at[idx_ref(i + 1)], bufs[(i + 1) % NBUF], gsem
                )
            gathers[i].wait()
            if i == 0:
                pcopy.wait()
            po = (i % spw) * SUB
            buf = bufs[i % NBUF]

            @plsc.parallel_loop(0, SUB)
            def row_body(r):
                for j in range(j_iters):
                    o = j * LANES
                    buf[r, pl.ds(o, LANES)] = (
                        buf[r, pl.ds(o, LANES)] * _SCALE
                        + pos_v[po + r, pl.ds(o, LANES)]
                    )
            stores[i] = pltpu.async_copy(
                buf, out_hbm.at[i // spw, pl.ds(t0 + po, SUB)], ssem
            )
        for i in range(max(0, nsub - NBUF), nsub):
            stores[i].wait()

    return k(x, pos, table)


def kernel(x, table):
    pos = jnp.asarray(_POS_NP)
    return _sc_embed(x, pos, table)


# speedup vs baseline: 1.1217x; 1.1217x over previous
import functools
import math

import jax
import jax.numpy as jnp
import numpy as np
from jax import lax
from jax.experimental import pallas as pl
from jax.experimental.pallas import tpu as pltpu
from jax.experimental.pallas import tpu_sc as plsc

D_MODEL = 768
POS_LEN = 2048
_SCALE = math.sqrt(float(D_MODEL))

NC, NS = 2, 16
NW = NC * NS
LANES = 16
TSUB = 8
NBUF = 3


def _positional_encoding_np(length, depth):
    half = depth / 2
    positions = np.arange(length)[:, np.newaxis]
    depths = np.arange(half)[np.newaxis, :] / half
    angle_rates = 1 / 10000 ** depths
    angle_rads = positions * angle_rates
    return np.concatenate(
        [np.sin(angle_rads), np.cos(angle_rads)], axis=-1
    ).astype(np.float32)


_POS_NP = _positional_encoding_np(POS_LEN, D_MODEL)


@functools.partial(jax.jit, static_argnames=())
def _sc_embed(x, pos, table):
    b_seq, t_seq = x.shape
    d = table.shape[1]
    t_per_w = t_seq // NW
    nsub = t_per_w // TSUB
    j_iters = d // LANES
    mesh = plsc.VectorSubcoreMesh(core_axis_name="c", subcore_axis_name="s")

    @functools.partial(
        pl.kernel,
        out_type=jax.ShapeDtypeStruct((b_seq, t_seq, d), jnp.float32),
        mesh=mesh,
        scratch_types=[
            pltpu.VMEM((b_seq, t_per_w), jnp.int32),
            pltpu.VMEM((t_per_w, d), jnp.float32),
        ]
        + [pltpu.VMEM((b_seq * TSUB, d), jnp.float32) for _ in range(NBUF)]
        + [
            pltpu.SemaphoreType.DMA,
            pltpu.SemaphoreType.DMA,
            pltpu.SemaphoreType.DMA,
        ],
    )
    def k(x_hbm, pos_hbm, table_hbm, out_hbm, idx_v, pos_v, *rest):
        bufs, (gsem, ssem, psem) = rest[:NBUF], rest[NBUF:]
        w = lax.axis_index("s") * NC + lax.axis_index("c")
        t0 = w * t_per_w
        for b in range(b_seq):
            pltpu.sync_copy(x_hbm.at[b, pl.ds(t0, t_per_w)], idx_v.at[b])
        pcopy = pltpu.async_copy(pos_hbm.at[pl.ds(t0, t_per_w)], pos_v, psem)

        def start_gathers(i):
            buf = bufs[i % NBUF]
            return [
                pltpu.async_copy(
                    table_hbm.at[idx_v.at[b, pl.ds(i * TSUB, TSUB)]],
                    buf.at[pl.ds(b * TSUB, TSUB)],
                    gsem,
                )
                for b in range(b_seq)
            ]

        gathers = [None] * nsub
        stores = [None] * nsub
        gathers[0] = start_gathers(0)
        for i in range(nsub):
            if i + 1 < nsub:
                if i + 1 >= NBUF:
                    for s in stores[i + 1 - NBUF]:
                        s.wait()
                gathers[i + 1] = start_gathers(i + 1)
            for g in gathers[i]:
                g.wait()
            if i == 0:
                pcopy.wait()
            po = i * TSUB
            buf = bufs[i % NBUF]

            @plsc.parallel_loop(0, j_iters)
            def col_body(j):
                o = pl.multiple_of(j * LANES, LANES)
                for r in range(TSUB):
                    pv = pos_v[po + r, pl.ds(o, LANES)]
                    for b in range(b_seq):
                        row = b * TSUB + r
                        buf[row, pl.ds(o, LANES)] = (
                            buf[row, pl.ds(o, LANES)] * _SCALE + pv
                        )

            stores[i] = [
                pltpu.async_copy(
                    buf.at[pl.ds(b * TSUB, TSUB)],
                    out_hbm.at[b, pl.ds(t0 + po, TSUB)],
                    ssem,
                )
                for b in range(b_seq)
            ]
        for i in range(max(0, nsub - NBUF), nsub):
            for s in stores[i]:
                s.wait()

    return k(x, pos, table)


def kernel(x, table):
    pos = jnp.asarray(_POS_NP)
    return _sc_embed(x, pos, table)


# speedup vs baseline: 1.1660x; 1.0394x over previous
import functools
import math

import jax
import jax.numpy as jnp
import numpy as np
from jax import lax
from jax.experimental import pallas as pl
from jax.experimental.pallas import tpu as pltpu
from jax.experimental.pallas import tpu_sc as plsc

D_MODEL = 768
POS_LEN = 2048
_SCALE = math.sqrt(float(D_MODEL))

NC, NS = 2, 16
NW = NC * NS
LANES = 16
TSUB = 8
NBUF = 3


def _positional_encoding_np(length, depth):
    half = depth / 2
    positions = np.arange(length)[:, np.newaxis]
    depths = np.arange(half)[np.newaxis, :] / half
    angle_rates = 1 / 10000 ** depths
    angle_rads = positions * angle_rates
    return np.concatenate(
        [np.sin(angle_rads), np.cos(angle_rads)], axis=-1
    ).astype(np.float32)


_POS_NP = _positional_encoding_np(POS_LEN, D_MODEL)


@functools.partial(jax.jit, static_argnames=())
def _sc_embed(x, pos, table):
    b_seq, t_seq = x.shape
    d = table.shape[1]
    t_per_w = t_seq // NW
    nsub = t_per_w // TSUB
    j_iters = d // LANES
    mesh = plsc.VectorSubcoreMesh(core_axis_name="c", subcore_axis_name="s")

    @functools.partial(
        pl.kernel,
        out_type=jax.ShapeDtypeStruct((b_seq, t_seq, d), jnp.float32),
        mesh=mesh,
        scratch_types=[
            pltpu.VMEM((b_seq, t_per_w), jnp.int32),
            pltpu.VMEM((t_per_w, d), jnp.float32),
        ]
        + [pltpu.VMEM((b_seq * TSUB, d), jnp.float32) for _ in range(NBUF)]
        + [
            pltpu.SemaphoreType.DMA,
            pltpu.SemaphoreType.DMA,
            pltpu.SemaphoreType.DMA,
        ],
    )
    def k(x_hbm, pos_hbm, table_hbm, out_hbm, idx_v, pos_v, *rest):
        bufs, (gsem, ssem, psem) = rest[:NBUF], rest[NBUF:]
        w = lax.axis_index("s") * NC + lax.axis_index("c")
        t0 = w * t_per_w
        icopies = [
            pltpu.async_copy(
                x_hbm.at[b, pl.ds(t0, t_per_w)], idx_v.at[b], psem
            )
            for b in range(b_seq)
        ]
        pcopy = pltpu.async_copy(pos_hbm.at[pl.ds(t0, t_per_w)], pos_v, psem)
        for c in icopies:
            c.wait()

        def start_gathers(i):
            buf = bufs[i % NBUF]
            return [
                pltpu.async_copy(
                    table_hbm.at[idx_v.at[b, pl.ds(i * TSUB, TSUB)]],
                    buf.at[pl.ds(b * TSUB, TSUB)],
                    gsem,
                )
                for b in range(b_seq)
            ]

        gathers = [None] * nsub
        stores = [None] * nsub
        gathers[0] = start_gathers(0)
        for i in range(nsub):
            if i + 1 < nsub:
                if i + 1 >= NBUF:
                    for s in stores[i + 1 - NBUF]:
                        s.wait()
                gathers[i + 1] = start_gathers(i + 1)
            for g in gathers[i]:
                g.wait()
            if i == 0:
                pcopy.wait()
            po = i * TSUB
            buf = bufs[i % NBUF]

            @plsc.parallel_loop(0, j_iters)
            def col_body(j):
                o = pl.multiple_of(j * LANES, LANES)
                for r in range(TSUB):
                    pv = pos_v[po + r, pl.ds(o, LANES)]
                    for b in range(b_seq):
                        row = b * TSUB + r
                        buf[row, pl.ds(o, LANES)] = (
                            buf[row, pl.ds(o, LANES)] * _SCALE + pv
                        )

            stores[i] = [
                pltpu.async_copy(
                    buf.at[pl.ds(b * TSUB, TSUB)],
                    out_hbm.at[b, pl.ds(t0 + po, TSUB)],
                    ssem,
                )
                for b in range(b_seq)
            ]
        for i in range(max(0, nsub - NBUF), nsub):
            for s in stores[i]:
                s.wait()

    return k(x, pos, table)


def kernel(x, table):
    pos = jnp.asarray(_POS_NP)
    return _sc_embed(x, pos, table)
